# baseline (speedup 1.0000x reference)
import functools

import jax
import jax.numpy as jnp
from jax import lax
from jax.experimental import pallas as pl
from jax.experimental.pallas import tpu as pltpu

B, S, D = 32, 2048, 256
TOPK = 3
BB = 2


def _routing_kernel(gw_r_ref, gw_i_ref, wg_ref, bg_ref, wa_t_ref, ba_ref,
                    idx_ref, euv_ref, ent_ref):
    flat = jnp.concatenate([gw_r_ref[...], gw_i_ref[...]], axis=1)
    gate_logit = jnp.sum(flat * wg_ref[...], axis=1, keepdims=True) + bg_ref[0, 0]
    write_gate = jax.nn.sigmoid(gate_logit)
    logits = jnp.dot(flat, wa_t_ref[...],
                     preferred_element_type=jnp.float32) + ba_ref[...]
    m = jnp.max(logits, axis=1, keepdims=True)
    e = jnp.exp(logits - m)
    ww = e / jnp.sum(e, axis=1, keepdims=True)
    ent = jnp.sum(-(ww * jnp.log(ww + 1e-10)), axis=1, keepdims=True)
    ent_ref[...] = jnp.sum(ent, axis=0, keepdims=True) * (1.0 / B)
    col = lax.broadcasted_iota(jnp.int32, (B, S), 1)
    work = ww
    idxs, vals = [], []
    for _ in range(TOPK):
        mx = jnp.max(work, axis=1, keepdims=True)
        first = jnp.min(jnp.where(work == mx, col, S), axis=1, keepdims=True)
        idxs.append(first)
        vals.append(mx)
        work = jnp.where(col == first, -jnp.inf, work)
    v = jnp.concatenate(vals, axis=1)
    scale = write_gate / (jnp.sum(v, axis=1, keepdims=True) + 1e-6)
    idx_ref[...] = jnp.concatenate(idxs, axis=1)
    euv_ref[...] = v * scale


def _stream_kernel(idx_ref, euv_ref,
                   q_r_ref, q_i_ref, g_r_ref, b_r_ref, g_i_ref, b_i_ref,
                   mem_r_ref, mem_i_ref,
                   read_r_ref, read_i_ref, next_r_ref, next_i_ref):
    step = pl.program_id(0)

    def _ln_row(x, gamma, beta):
        mu = jnp.mean(x, axis=1, keepdims=True)
        xc = x - mu
        var = jnp.mean(xc * xc, axis=1, keepdims=True)
        return xc * lax.rsqrt(var + 1e-5) * gamma + beta

    def _ln_dense(x, gamma, beta):
        mu = jnp.mean(x, axis=1, keepdims=True)
        var = jnp.mean(x * x, axis=1, keepdims=True) - mu * mu
        rg = lax.rsqrt(var + 1e-5)
        h = -(mu * rg)
        return (x * rg + h) * gamma + beta

    for ib in range(BB):
        mem_r = mem_r_ref[ib]
        mem_i = mem_i_ref[ib]
        q_r = q_r_ref[ib]
        q_i = q_i_ref[ib]

        sim = jnp.sum(mem_r * q_r + mem_i * q_i, axis=1, keepdims=True)
        p = jnp.exp(sim - jnp.max(sim))
        inv_l = 1.0 / jnp.sum(p)
        read_r_ref[ib] = jnp.sum(p * mem_r, axis=0, keepdims=True) * inv_l
        read_i_ref[ib] = jnp.sum(p * mem_i, axis=0, keepdims=True) * inv_l

        next_r_ref[ib] = _ln_dense(mem_r, g_r_ref[...], b_r_ref[...])
        next_i_ref[ib] = _ln_dense(mem_i, g_i_ref[...], b_i_ref[...])

        for k in range(TOPK):
            i = idx_ref[step * BB + ib, k]
            e = euv_ref[step * BB + ib, k]
            row_r = mem_r_ref[ib, pl.ds(i, 1), :]
            row_i = mem_i_ref[ib, pl.ds(i, 1), :]
            nr = row_r + e * (q_r - row_r)
            ni = row_i + e * (q_i - row_i)
            next_r_ref[ib, pl.ds(i, 1), :] = _ln_row(nr, g_r_ref[...], b_r_ref[...])
            next_i_ref[ib, pl.ds(i, 1), :] = _ln_row(ni, g_i_ref[...], b_i_ref[...])


@functools.partial(jax.jit, static_argnames=("interpret",))
def kernel(gw_state_real, gw_state_imag, prev_mem_real, prev_mem_imag,
           Wg, bg, Wa, ba, gamma_r, beta_r, gamma_i, beta_i, interpret=False):
    f32 = jnp.float32
    idx, euv, ent = pl.pallas_call(
        _routing_kernel,
        out_shape=(jax.ShapeDtypeStruct((B, TOPK), jnp.int32),
                   jax.ShapeDtypeStruct((B, TOPK), f32),
                   jax.ShapeDtypeStruct((1, 1), f32)),
        interpret=interpret,
    )(gw_state_real, gw_state_imag, Wg, bg.reshape(1, 1), Wa.T,
      ba.reshape(1, S))

    q_r = gw_state_real.reshape(B, 1, D)
    q_i = gw_state_imag.reshape(B, 1, D)

    grid_spec = pltpu.PrefetchScalarGridSpec(
        num_scalar_prefetch=2,
        grid=(B // BB,),
        in_specs=[
            pl.BlockSpec((BB, 1, D), lambda b, *_: (b, 0, 0)),
            pl.BlockSpec((BB, 1, D), lambda b, *_: (b, 0, 0)),
            pl.BlockSpec((1, D), lambda b, *_: (0, 0)),
            pl.BlockSpec((1, D), lambda b, *_: (0, 0)),
            pl.BlockSpec((1, D), lambda b, *_: (0, 0)),
            pl.BlockSpec((1, D), lambda b, *_: (0, 0)),
            pl.BlockSpec((BB, S, D), lambda b, *_: (b, 0, 0)),
            pl.BlockSpec((BB, S, D), lambda b, *_: (b, 0, 0)),
        ],
        out_specs=[
            pl.BlockSpec((BB, 1, D), lambda b, *_: (b, 0, 0)),
            pl.BlockSpec((BB, 1, D), lambda b, *_: (b, 0, 0)),
            pl.BlockSpec((BB, S, D), lambda b, *_: (b, 0, 0)),
            pl.BlockSpec((BB, S, D), lambda b, *_: (b, 0, 0)),
        ],
    )
    read_r, read_i, next_r, next_i = pl.pallas_call(
        _stream_kernel,
        grid_spec=grid_spec,
        out_shape=(jax.ShapeDtypeStruct((B, 1, D), f32),
                   jax.ShapeDtypeStruct((B, 1, D), f32),
                   jax.ShapeDtypeStruct((B, S, D), f32),
                   jax.ShapeDtypeStruct((B, S, D), f32)),
        interpret=interpret,
    )(idx, euv, q_r, q_i, gamma_r.reshape(1, D), beta_r.reshape(1, D),
      gamma_i.reshape(1, D), beta_i.reshape(1, D), prev_mem_real, prev_mem_imag)

    return (read_r.reshape(B, D), read_i.reshape(B, D), next_r, next_i,
            ent.reshape(()))
